# baseline (speedup 1.0000x reference)
import functools

import jax
import jax.numpy as jnp
from jax import lax
from jax.experimental import pallas as pl
from jax.experimental.pallas import tpu as pltpu
from jax.experimental.pallas import tpu_sc as plsc

N = 10000
E = 320000
D = 128
CW = 32

NC = 2
NS = 16
NW = NC * NS
EPW = E // NW
K = 80
NCHUNK = EPW // K
NP = 10240
RPS = NP // NS

_MESH = dict(core_axis_name="c", subcore_axis_name="s", num_cores=NC,
             num_subcores=NS)


def _sc_segment_sum(src, dst, x, z2d):
  @functools.partial(
      pl.kernel,
      mesh=plsc.VectorSubcoreMesh(**_MESH),
      out_type=jax.ShapeDtypeStruct((NC * NP, D), jnp.float32),
      scratch_types=[
          pltpu.VMEM_SHARED((NP, D), jnp.float32),
          pltpu.VMEM((EPW,), jnp.int32),
          pltpu.VMEM((K,), jnp.int32),
          pltpu.VMEM((K,), jnp.int32),
          pltpu.VMEM((K, D), jnp.float32),
          pltpu.VMEM((K, D), jnp.float32),
          pltpu.SemaphoreType.DMA,
          pltpu.SemaphoreType.DMA,
          pltpu.SemaphoreType.DMA,
          pltpu.SemaphoreType.DMA,
      ],
  )
  def body(src_hbm, dst_hbm, x_hbm, z2d_hbm, sum_out, acc_sh,
           src_all, dstb0, dstb1, rows0, rows1, sg0, sg1, sd0, sd1):
    c = lax.axis_index("c")
    s = lax.axis_index("s")
    wid = s * NC + c

    srow0 = pl.multiple_of(s * RPS, 8)
    row0 = pl.multiple_of(c * NP + s * RPS, 8)
    base = pl.multiple_of(wid * EPW, 8)
    pltpu.sync_copy(z2d_hbm, acc_sh.at[pl.ds(srow0, RPS), :])
    pltpu.sync_copy(src_hbm.at[pl.ds(base, EPW)], src_all)
    plsc.subcore_barrier()

    def gidx(j):
      return src_all.at[pl.ds(j * K, K)]

    def dslice(j):
      return dst_hbm.at[pl.ds(base + j * K, K)]

    pltpu.async_copy(dslice(0), dstb0, sd0)
    pltpu.async_copy(x_hbm.at[gidx(0)], rows0, sg0)

    def pair(p, carry):
      j0 = 2 * p
      dd1 = pltpu.async_copy(dslice(j0 + 1), dstb1, sd1)
      dg1 = pltpu.async_copy(x_hbm.at[gidx(j0 + 1)], rows1, sg1)
      pltpu.make_async_copy(dslice(j0), dstb0, sd0).wait()
      pltpu.make_async_copy(x_hbm.at[gidx(j0)], rows0, sg0).wait()
      pltpu.sync_copy(rows0, acc_sh.at[dstb0], add=True)
      pltpu.async_copy(dslice(j0 + 2), dstb0, sd0)
      pltpu.async_copy(x_hbm.at[gidx(j0 + 2)], rows0, sg0)
      dd1.wait()
      dg1.wait()
      pltpu.sync_copy(rows1, acc_sh.at[dstb1], add=True)
      return carry

    lax.fori_loop(0, (NCHUNK - 1) // 2, pair, 0)
    pltpu.make_async_copy(dslice(NCHUNK - 1), dstb0, sd0).wait()
    pltpu.make_async_copy(x_hbm.at[gidx(NCHUNK - 1)], rows0, sg0).wait()
    pltpu.sync_copy(rows0, acc_sh.at[dstb0], add=True)
    plsc.subcore_barrier()

    pltpu.sync_copy(acc_sh.at[pl.ds(srow0, RPS), :],
                    sum_out.at[pl.ds(row0, RPS), :])

  return body(src, dst, x, z2d)


CCW = 8


def _sc_degree(dst, zc, ones_h, dep):
  @functools.partial(
      pl.kernel,
      mesh=plsc.VectorSubcoreMesh(**_MESH),
      out_type=jax.ShapeDtypeStruct((NC * NP, D), jnp.float32),
      scratch_types=[
          pltpu.VMEM_SHARED((NP, D), jnp.float32),
          pltpu.VMEM((K,), jnp.int32),
          pltpu.VMEM((K,), jnp.int32),
          pltpu.VMEM((K, D), jnp.float32),
          pltpu.SemaphoreType.DMA,
          pltpu.SemaphoreType.DMA,
      ],
  )
  def body(dst_hbm, zc_hbm, ones_hbm, dep_hbm, cnt_out, cnt_sh, dstb0,
           dstb1, ones_v, sd0, sd1):
    c = lax.axis_index("c")
    s = lax.axis_index("s")
    wid = s * NC + c

    srow0 = pl.multiple_of(s * RPS, 8)
    row0 = pl.multiple_of(c * NP + s * RPS, 8)
    base = pl.multiple_of(wid * EPW, 8)
    pltpu.sync_copy(zc_hbm, cnt_sh.at[pl.ds(srow0, RPS), :])
    pltpu.sync_copy(ones_hbm, ones_v)
    plsc.subcore_barrier()

    def dslice(j):
      return dst_hbm.at[pl.ds(base + j * K, K)]

    pltpu.async_copy(dslice(0), dstb0, sd0)

    def pair(p, carry):
      j0 = 2 * p
      dd1 = pltpu.async_copy(dslice(j0 + 1), dstb1, sd1)
      pltpu.make_async_copy(dslice(j0), dstb0, sd0).wait()
      pltpu.sync_copy(ones_v, cnt_sh.at[dstb0], add=True)
      pltpu.async_copy(dslice(j0 + 2), dstb0, sd0)
      dd1.wait()
      pltpu.sync_copy(ones_v, cnt_sh.at[dstb1], add=True)
      return carry

    lax.fori_loop(0, (NCHUNK - 1) // 2, pair, 0)
    pltpu.make_async_copy(dslice(NCHUNK - 1), dstb0, sd0).wait()
    pltpu.sync_copy(ones_v, cnt_sh.at[dstb0], add=True)
    plsc.subcore_barrier()

    pltpu.sync_copy(cnt_sh.at[pl.ds(srow0, RPS), :],
                    cnt_out.at[pl.ds(row0, RPS), :])

  return body(dst, zc, ones_h, dep)


def _tc_sage(x, sum_parts, cnt_parts, w1t, w2t, b2):
  R = 1000

  def body(x_ref, sp_ref, cp_ref, w1_ref, w2_ref, b_ref, o_ref):
    xb = x_ref[...]
    summed = sp_ref[0] + sp_ref[1]
    cnt = cp_ref[0, :, 0:1] + cp_ref[1, :, 0:1]
    mean = summed / jnp.maximum(cnt, 1.0)
    agg = jnp.where(cnt > 0.0, mean, xb)
    h = (jnp.dot(xb, w1_ref[...], preferred_element_type=jnp.float32)
         + jnp.dot(agg, w2_ref[...], preferred_element_type=jnp.float32)
         + b_ref[...])
    o_ref[...] = jnp.maximum(h, 0.0)

  return pl.pallas_call(
      body,
      grid=(N // R,),
      in_specs=[
          pl.BlockSpec((R, D), lambda i: (i, 0)),
          pl.BlockSpec((NC, R, D), lambda i: (0, i, 0)),
          pl.BlockSpec((NC, R, D), lambda i: (0, i, 0)),
          pl.BlockSpec((D, D), lambda i: (0, 0)),
          pl.BlockSpec((D, D), lambda i: (0, 0)),
          pl.BlockSpec((1, D), lambda i: (0, 0)),
      ],
      out_specs=pl.BlockSpec((R, D), lambda i: (i, 0)),
      out_shape=jax.ShapeDtypeStruct((N, D), jnp.float32),
  )(x, sum_parts, cnt_parts, w1t, w2t, b2)


def kernel(x, edge_index, W, b):
  src = edge_index[0].astype(jnp.int32)
  dst = edge_index[1].astype(jnp.int32)
  z2d = jnp.zeros((RPS, D), jnp.float32)
  sum_flat = _sc_segment_sum(src, dst, x, z2d)
  ones2d = jnp.ones((K, D), jnp.float32)
  cnt_flat = _sc_degree(dst, z2d, ones2d, sum_flat)
  sum_parts = sum_flat.reshape(NC, NP, D)
  cnt_parts = cnt_flat.reshape(NC, NP, D)
  w1t = W[:, :D].T
  w2t = W[:, D:].T
  b2 = b[None, :]
  return _tc_sage(x, sum_parts, cnt_parts, w1t, w2t, b2)


# speedup vs baseline: 1.0058x; 1.0058x over previous
import functools

import jax
import jax.numpy as jnp
from jax import lax
from jax.experimental import pallas as pl
from jax.experimental.pallas import tpu as pltpu
from jax.experimental.pallas import tpu_sc as plsc

N = 10000
E = 320000
D = 128
CW = 32

NC = 2
NS = 16
NW = NC * NS
EPW = E // NW
K = 80
NCHUNK = EPW // K
NP = 10112
RPS = NP // NS

_MESH = dict(core_axis_name="c", subcore_axis_name="s", num_cores=NC,
             num_subcores=NS)


def _sc_segment_sum(src, dst, x, z2d):
  @functools.partial(
      pl.kernel,
      mesh=plsc.VectorSubcoreMesh(**_MESH),
      out_type=jax.ShapeDtypeStruct((NC * NP, D), jnp.float32),
      scratch_types=[
          pltpu.VMEM_SHARED((NP, D), jnp.float32),
          pltpu.VMEM((K,), jnp.int32),
          pltpu.VMEM((K,), jnp.int32),
          pltpu.VMEM((K,), jnp.int32),
          pltpu.VMEM((K,), jnp.int32),
          pltpu.VMEM((K,), jnp.int32),
          pltpu.VMEM((K,), jnp.int32),
          pltpu.VMEM((K, D), jnp.float32),
          pltpu.VMEM((K, D), jnp.float32),
          pltpu.VMEM((K, D), jnp.float32),
          pltpu.SemaphoreType.DMA,
          pltpu.SemaphoreType.DMA,
          pltpu.SemaphoreType.DMA,
          pltpu.SemaphoreType.DMA,
          pltpu.SemaphoreType.DMA,
          pltpu.SemaphoreType.DMA,
      ],
  )
  def body(src_hbm, dst_hbm, x_hbm, z2d_hbm, sum_out, acc_sh,
           sb0, sb1, sb2, db0, db1, db2, r0, r1, r2,
           si0, si1, si2, sg0, sg1, sg2):
    c = lax.axis_index("c")
    s = lax.axis_index("s")
    wid = s * NC + c
    sb = (sb0, sb1, sb2)
    db = (db0, db1, db2)
    rows = (r0, r1, r2)
    si = (si0, si1, si2)
    sg = (sg0, sg1, sg2)

    srow0 = pl.multiple_of(s * RPS, 8)
    row0 = pl.multiple_of(c * NP + s * RPS, 8)
    base = pl.multiple_of(wid * EPW, 8)
    pltpu.sync_copy(z2d_hbm, acc_sh.at[pl.ds(srow0, RPS), :])
    plsc.subcore_barrier()

    def sslice(j):
      return src_hbm.at[pl.ds(base + j * K, K)]

    def dslice(j):
      return dst_hbm.at[pl.ds(base + j * K, K)]

    def start_idx(j, r):
      pltpu.async_copy(sslice(j), sb[r], si[r])
      pltpu.async_copy(dslice(j), db[r], si[r])

    def wait_idx(j, r):
      pltpu.make_async_copy(sslice(j), sb[r], si[r]).wait()
      pltpu.make_async_copy(dslice(j), db[r], si[r]).wait()

    def start_gather(r):
      pltpu.async_copy(x_hbm.at[sb[r]], rows[r], sg[r])

    def wait_gather(r):
      pltpu.make_async_copy(x_hbm.at[sb[r]], rows[r], sg[r]).wait()

    def scatter(r):
      pltpu.sync_copy(rows[r], acc_sh.at[db[r]], add=True)

    start_idx(0, 0)
    start_idx(1, 1)
    start_idx(2, 2)
    wait_idx(0, 0)
    start_gather(0)
    wait_idx(1, 1)
    start_gather(1)

    def group(p, carry):
      j0 = 3 * p
      for i in range(3):
        r = i
        r2 = (i + 2) % 3
        wait_gather(r)
        scatter(r)
        start_idx(j0 + i + 3, r)
        wait_idx(j0 + i + 2, r2)
        start_gather(r2)
      return carry

    lax.fori_loop(0, (NCHUNK - 5) // 3, group, 0)

    jb = NCHUNK - 5
    for i in range(5):
      j = jb + i
      r = j % 3
      wait_gather(r)
      scatter(r)
      if j + 3 <= NCHUNK - 1:
        start_idx(j + 3, r)
      if j + 2 <= NCHUNK - 1:
        r2 = (j + 2) % 3
        wait_idx(j + 2, r2)
        start_gather(r2)
    plsc.subcore_barrier()

    pltpu.sync_copy(acc_sh.at[pl.ds(srow0, RPS), :],
                    sum_out.at[pl.ds(row0, RPS), :])

  return body(src, dst, x, z2d)


CCW = 8


def _sc_degree(dst, zc, ones_h, dep):
  @functools.partial(
      pl.kernel,
      mesh=plsc.VectorSubcoreMesh(**_MESH),
      out_type=jax.ShapeDtypeStruct((NC * NP, D), jnp.float32),
      scratch_types=[
          pltpu.VMEM_SHARED((NP, D), jnp.float32),
          pltpu.VMEM((K,), jnp.int32),
          pltpu.VMEM((K,), jnp.int32),
          pltpu.VMEM((K, D), jnp.float32),
          pltpu.SemaphoreType.DMA,
          pltpu.SemaphoreType.DMA,
      ],
  )
  def body(dst_hbm, zc_hbm, ones_hbm, dep_hbm, cnt_out, cnt_sh, dstb0,
           dstb1, ones_v, sd0, sd1):
    c = lax.axis_index("c")
    s = lax.axis_index("s")
    wid = s * NC + c

    srow0 = pl.multiple_of(s * RPS, 8)
    row0 = pl.multiple_of(c * NP + s * RPS, 8)
    base = pl.multiple_of(wid * EPW, 8)
    pltpu.sync_copy(zc_hbm, cnt_sh.at[pl.ds(srow0, RPS), :])
    pltpu.sync_copy(ones_hbm, ones_v)
    plsc.subcore_barrier()

    def dslice(j):
      return dst_hbm.at[pl.ds(base + j * K, K)]

    pltpu.async_copy(dslice(0), dstb0, sd0)

    def pair(p, carry):
      j0 = 2 * p
      dd1 = pltpu.async_copy(dslice(j0 + 1), dstb1, sd1)
      pltpu.make_async_copy(dslice(j0), dstb0, sd0).wait()
      pltpu.sync_copy(ones_v, cnt_sh.at[dstb0], add=True)
      pltpu.async_copy(dslice(j0 + 2), dstb0, sd0)
      dd1.wait()
      pltpu.sync_copy(ones_v, cnt_sh.at[dstb1], add=True)
      return carry

    lax.fori_loop(0, (NCHUNK - 1) // 2, pair, 0)
    pltpu.make_async_copy(dslice(NCHUNK - 1), dstb0, sd0).wait()
    pltpu.sync_copy(ones_v, cnt_sh.at[dstb0], add=True)
    plsc.subcore_barrier()

    pltpu.sync_copy(cnt_sh.at[pl.ds(srow0, RPS), :],
                    cnt_out.at[pl.ds(row0, RPS), :])

  return body(dst, zc, ones_h, dep)


def _tc_sage(x, sum_parts, cnt_parts, w1t, w2t, b2):
  R = 1000

  def body(x_ref, sp_ref, cp_ref, w1_ref, w2_ref, b_ref, o_ref):
    xb = x_ref[...]
    summed = sp_ref[0] + sp_ref[1]
    cnt = cp_ref[0, :, 0:1] + cp_ref[1, :, 0:1]
    mean = summed / jnp.maximum(cnt, 1.0)
    agg = jnp.where(cnt > 0.0, mean, xb)
    h = (jnp.dot(xb, w1_ref[...], preferred_element_type=jnp.float32)
         + jnp.dot(agg, w2_ref[...], preferred_element_type=jnp.float32)
         + b_ref[...])
    o_ref[...] = jnp.maximum(h, 0.0)

  return pl.pallas_call(
      body,
      grid=(N // R,),
      in_specs=[
          pl.BlockSpec((R, D), lambda i: (i, 0)),
          pl.BlockSpec((NC, R, D), lambda i: (0, i, 0)),
          pl.BlockSpec((NC, R, D), lambda i: (0, i, 0)),
          pl.BlockSpec((D, D), lambda i: (0, 0)),
          pl.BlockSpec((D, D), lambda i: (0, 0)),
          pl.BlockSpec((1, D), lambda i: (0, 0)),
      ],
      out_specs=pl.BlockSpec((R, D), lambda i: (i, 0)),
      out_shape=jax.ShapeDtypeStruct((N, D), jnp.float32),
  )(x, sum_parts, cnt_parts, w1t, w2t, b2)


def kernel(x, edge_index, W, b):
  src = edge_index[0].astype(jnp.int32)
  dst = edge_index[1].astype(jnp.int32)
  z2d = jnp.zeros((RPS, D), jnp.float32)
  sum_flat = _sc_segment_sum(src, dst, x, z2d)
  ones2d = jnp.ones((K, D), jnp.float32)
  cnt_flat = _sc_degree(dst, z2d, ones2d, sum_flat)
  sum_parts = sum_flat.reshape(NC, NP, D)
  cnt_parts = cnt_flat.reshape(NC, NP, D)
  w1t = W[:, :D].T
  w2t = W[:, D:].T
  b2 = b[None, :]
  return _tc_sage(x, sum_parts, cnt_parts, w1t, w2t, b2)
